# baseline (speedup 1.0000x reference)
import functools

import jax
import jax.numpy as jnp
from jax import lax
from jax.experimental import pallas as pl
from jax.experimental.pallas import tpu as pltpu
from jax.experimental.pallas import tpu_sc as plsc

BATCH = 16384
EMBED_DIM = 128
CHUNK = 128
NUM_WORKERS = 32
ROWS_PER_WORKER = BATCH // NUM_WORKERS
CHUNKS_PER_WORKER = ROWS_PER_WORKER // CHUNK
GROUPS_PER_CHUNK = CHUNK // 16


def _sc_body(user_ids, item_ids, user_table, item_table, bias, out,
             idx_u, idx_i, u0, u1, i0, i1, bias_v, out_v, stage,
             sem0, sem1, semi):
    wid = lax.axis_index("s") * 2 + lax.axis_index("c")
    base = wid * ROWS_PER_WORKER

    ubufs = (u0, u1)
    ibufs = (i0, i1)
    sems = (sem0, sem1)

    def stage_idx(j):
        return (
            pltpu.async_copy(user_ids.at[pl.ds(base + j * CHUNK, CHUNK)],
                             idx_u.at[j], semi),
            pltpu.async_copy(item_ids.at[pl.ds(base + j * CHUNK, CHUNK)],
                             idx_i.at[j], semi),
        )

    def start_gathers(j):
        slot = j % 2
        sem = sems[slot]
        return (
            pltpu.async_copy(user_table.at[idx_u.at[j]], ubufs[slot], sem),
            pltpu.async_copy(item_table.at[idx_i.at[j]], ibufs[slot], sem),
        )

    for h in stage_idx(0):
        h.wait()
    lane = lax.iota(jnp.int32, 16)
    pending = start_gathers(0)
    idx_copies = [pltpu.async_copy(bias.at[pl.ds(base, ROWS_PER_WORKER)],
                                   bias_v, semi)]
    for j in range(1, CHUNKS_PER_WORKER):
        idx_copies.extend(stage_idx(j))
    for h in idx_copies:
        h.wait()

    for j in range(CHUNKS_PER_WORKER):
        for h in pending:
            h.wait()
        if j + 1 < CHUNKS_PER_WORKER:
            pending = start_gathers(j + 1)
        slot = j % 2
        U = ubufs[slot]
        I = ibufs[slot]

        def group_body(g, _, j=j, U=U, I=I):
            for r in range(16):
                row = g * 16 + r
                acc = U[row, pl.ds(0, 16)] * I[row, pl.ds(0, 16)]
                for k in range(1, EMBED_DIM // 16):
                    acc = acc + (U[row, pl.ds(k * 16, 16)]
                                 * I[row, pl.ds(k * 16, 16)])
                stage[r, pl.ds(0, 16)] = acc
            res = plsc.load_gather(stage, [lane, jnp.zeros((16,), jnp.int32)])
            cvec = jnp.full((16,), 1, jnp.int32)
            for c in range(1, 16):
                res = res + plsc.load_gather(stage, [lane, cvec])
                if c < 15:
                    cvec = cvec + 1
            res = res + bias_v[pl.ds(j * CHUNK + g * 16, 16)]
            out_v[pl.ds(j * CHUNK + g * 16, 16)] = res
            return 0

        lax.fori_loop(0, GROUPS_PER_CHUNK, group_body, 0)

    pltpu.sync_copy(out_v, out.at[pl.ds(base, ROWS_PER_WORKER)])


@jax.jit
def _sc_call(user_ids, item_ids, user_table, item_table, bias):
    mesh = plsc.VectorSubcoreMesh(core_axis_name="c", subcore_axis_name="s")
    f = functools.partial(
        pl.kernel,
        out_type=jax.ShapeDtypeStruct((BATCH,), jnp.float32),
        mesh=mesh,
        compiler_params=pltpu.CompilerParams(needs_layout_passes=False),
        scratch_types=[
            pltpu.VMEM((CHUNKS_PER_WORKER, CHUNK), jnp.int32),
            pltpu.VMEM((CHUNKS_PER_WORKER, CHUNK), jnp.int32),
            pltpu.VMEM((CHUNK, EMBED_DIM), jnp.float32),
            pltpu.VMEM((CHUNK, EMBED_DIM), jnp.float32),
            pltpu.VMEM((CHUNK, EMBED_DIM), jnp.float32),
            pltpu.VMEM((CHUNK, EMBED_DIM), jnp.float32),
            pltpu.VMEM((ROWS_PER_WORKER,), jnp.float32),
            pltpu.VMEM((ROWS_PER_WORKER,), jnp.float32),
            pltpu.VMEM((16, 17), jnp.float32),
            pltpu.SemaphoreType.DMA,
            pltpu.SemaphoreType.DMA,
            pltpu.SemaphoreType.DMA,
        ],
    )(_sc_body)
    return f(user_ids, item_ids, user_table, item_table, bias)


def kernel(user_ids, item_ids, user_table, item_table, user_bias, item_bias):
    uids = user_ids.astype(jnp.int32)
    iids = item_ids.astype(jnp.int32)
    bias = (jnp.take(user_bias, uids, axis=0)
            + jnp.take(item_bias, iids, axis=0)).reshape(-1)
    out = _sc_call(uids, iids, user_table, item_table, bias)
    return out.reshape(BATCH, 1)
